# Initial kernel scaffold; baseline (speedup 1.0000x reference)
import jax
import jax.numpy as jnp
from jax.experimental import pallas as pl


def kernel(x, edge_index, W1, a1_src, a1_dst, b1, W2, a2_src, a2_dst, b2, W3, a3_src, a3_dst, b3):
    raise NotImplementedError("write your pallas kernel here")


# speedup vs baseline: 1.0917x; 1.0917x over previous
import jax
import jax.numpy as jnp
from jax.experimental import pallas as pl


def _identity_kernel(x_ref, o_ref):
    o_ref[...] = x_ref[...]


def _gat_conv(x, edge_index, W, a_src, a_dst, b):
    N = x.shape[0]
    loops = jnp.arange(N, dtype=edge_index.dtype)
    src = jnp.concatenate([edge_index[0], loops])
    dst = jnp.concatenate([edge_index[1], loops])
    h = x @ W
    alpha_s = h @ a_src
    alpha_d = h @ a_dst
    e = jax.nn.leaky_relu(alpha_s[src] + alpha_d[dst], negative_slope=0.2)
    m = jax.ops.segment_max(e, dst, num_segments=N)
    m = jnp.where(jnp.isfinite(m), m, 0.0)
    ex = jnp.exp(e - m[dst])
    den = jax.ops.segment_sum(ex, dst, num_segments=N)
    alpha = ex / (den[dst] + 1e-16)
    out = jax.ops.segment_sum(h[src] * alpha[:, None], dst, num_segments=N)
    return out + b


def kernel(x, edge_index, W1, a1_src, a1_dst, b1, W2, a2_src, a2_dst, b2, W3, a3_src, a3_dst, b3):
    x = pl.pallas_call(
        _identity_kernel,
        out_shape=jax.ShapeDtypeStruct(x.shape, x.dtype),
    )(x)
    h = _gat_conv(x, edge_index, W1, a1_src, a1_dst, b1)
    h = h * jax.nn.sigmoid(h)
    h = _gat_conv(h, edge_index, W2, a2_src, a2_dst, b2)
    h = h * jax.nn.sigmoid(h)
    h = _gat_conv(h, edge_index, W3, a3_src, a3_dst, b3)
    return jax.nn.log_softmax(h, axis=1)


# speedup vs baseline: 39.1841x; 35.8929x over previous
import functools

import jax
import jax.numpy as jnp
from jax import lax
from jax.experimental import pallas as pl
from jax.experimental.pallas import tpu as pltpu
from jax.experimental.pallas import tpu_sc as plsc

NC = 2
NS = 16
NW = NC * NS
B = 80
NP = 10240

_LEAKY = 0.2


def _leaky(v):
    return jnp.maximum(v, v * _LEAKY)


def _tc_pre_body(x_ref, w_ref, asrc_ref, adst_ref, h_ref, sn_ref):
    h = jnp.dot(x_ref[...], w_ref[...], preferred_element_type=jnp.float32)
    h_ref[...] = h
    a_s = jnp.dot(h, asrc_ref[...], preferred_element_type=jnp.float32)
    a_d = jnp.dot(h, adst_ref[...], preferred_element_type=jnp.float32)
    amax = jnp.max(a_s)
    c = _leaky(amax + a_d)
    sn_ref[...] = jnp.concatenate(
        [a_s, a_d, c, jnp.zeros_like(a_s)], axis=1)


def _tc_pre(x, w, a_src, a_dst):
    n = x.shape[0]
    dn = w.shape[1]
    return pl.pallas_call(
        _tc_pre_body,
        out_shape=[
            jax.ShapeDtypeStruct((n, dn), jnp.float32),
            jax.ShapeDtypeStruct((n, 4), jnp.float32),
        ],
    )(x, w, a_src.reshape(dn, 1), a_dst.reshape(dn, 1))


def _tc_norm(acc_ref, den_ref, h_ref, sn_ref, b_ref):
    a_s = sn_ref[:, 0:1]
    a_d = sn_ref[:, 1:2]
    c = sn_ref[:, 2:3]
    exs = jnp.exp(_leaky(a_s + a_d) - c)
    num = acc_ref[0] + acc_ref[1] + exs * h_ref[...]
    dent = den_ref[0] + den_ref[1] + exs + 1e-16
    return num / dent + b_ref[...]


def _tc_mid_body(acc_ref, den_ref, h_ref, sn_ref, b_ref, w_ref, asrc_ref,
                 adst_ref, h2_ref, sn2_ref):
    o = _tc_norm(acc_ref, den_ref, h_ref, sn_ref, b_ref)
    g = o * jax.nn.sigmoid(o)
    h2 = jnp.dot(g, w_ref[...], preferred_element_type=jnp.float32)
    h2_ref[...] = h2
    a_s = jnp.dot(h2, asrc_ref[...], preferred_element_type=jnp.float32)
    a_d = jnp.dot(h2, adst_ref[...], preferred_element_type=jnp.float32)
    amax = jnp.max(a_s)
    c = _leaky(amax + a_d)
    sn2_ref[...] = jnp.concatenate(
        [a_s, a_d, c, jnp.zeros_like(a_s)], axis=1)


def _tc_mid(acc, den, h, sn, b, w, a_src, a_dst):
    n, d = h.shape
    dn = w.shape[1]
    return pl.pallas_call(
        _tc_mid_body,
        out_shape=[
            jax.ShapeDtypeStruct((n, dn), jnp.float32),
            jax.ShapeDtypeStruct((n, 4), jnp.float32),
        ],
    )(acc, den, h, sn, b.reshape(1, d), w,
      a_src.reshape(dn, 1), a_dst.reshape(dn, 1))


def _tc_final_body(acc_ref, den_ref, h_ref, sn_ref, b_ref, out_ref):
    o = _tc_norm(acc_ref, den_ref, h_ref, sn_ref, b_ref)
    mx = jnp.max(o, axis=1, keepdims=True)
    lse = jnp.log(jnp.sum(jnp.exp(o - mx), axis=1, keepdims=True)) + mx
    out_ref[...] = o - lse


def _tc_final(acc, den, h, sn, b):
    n, d = h.shape
    return pl.pallas_call(
        _tc_final_body,
        out_shape=jax.ShapeDtypeStruct((n, d), jnp.float32),
    )(acc, den, h, sn, b.reshape(1, d))


def _sc_edge_body(nblk, d, rpt, h_hbm, sn_hbm, edges_hbm, acc_hbm, den_hbm,
                  src_v, dst_v, rowbuf, asb, adb, cb, exbuf, zden_v,
                  as_sh, ad_sh, c_sh, acc_sh, den_sh, sem):
    cid = lax.axis_index("c")
    sid = lax.axis_index("s")
    wid = cid * NS + sid
    base = sid * rpt

    @pl.when(sid == 0)
    def _stage_tables():
        pltpu.sync_copy(sn_hbm.at[0], as_sh)
        pltpu.sync_copy(sn_hbm.at[1], ad_sh)
        pltpu.sync_copy(sn_hbm.at[2], c_sh)

    pltpu.sync_copy(edges_hbm.at[0, wid], src_v)
    pltpu.sync_copy(edges_hbm.at[1, wid], dst_v)

    zero16 = jnp.zeros((16,), jnp.float32)

    def _zrow(r, _):
        for q in range(d // 16):
            rowbuf[r, pl.ds(q * 16, 16)] = zero16
        return 0

    lax.fori_loop(0, B, _zrow, 0)

    def _zden(r, _):
        zden_v[pl.ds(r * 16, 16)] = zero16
        return 0

    lax.fori_loop(0, rpt // 16, _zden, 0)

    for t in range(rpt // B):
        pltpu.sync_copy(rowbuf, acc_sh.at[pl.ds(base + t * B, B)])
    pltpu.sync_copy(zden_v, den_sh.at[pl.ds(base, rpt)])
    plsc.subcore_barrier()

    def _blk(j, _):
        gat = pltpu.async_copy(h_hbm.at[src_v.at[j]], rowbuf, sem)
        pltpu.sync_copy(as_sh.at[src_v.at[j]], asb)
        pltpu.sync_copy(ad_sh.at[dst_v.at[j]], adb)
        pltpu.sync_copy(c_sh.at[dst_v.at[j]], cb)
        gat.wait()

        for k in range(B // 16):
            as16 = asb[pl.ds(k * 16, 16)]
            ad16 = adb[pl.ds(k * 16, 16)]
            c16 = cb[pl.ds(k * 16, 16)]
            ex16 = jnp.exp(_leaky(as16 + ad16) - c16)
            exbuf[pl.ds(k * 16, 16)] = ex16

        def _scale(g, _):
            ex16 = exbuf[pl.ds(g * 16, 16)]
            for r in range(16):
                s = ex16[r]
                row = g * 16 + r
                for q in range(d // 16):
                    rowbuf[row, pl.ds(q * 16, 16)] = (
                        rowbuf[row, pl.ds(q * 16, 16)] * s)
            return 0

        lax.fori_loop(0, B // 16, _scale, 0)

        pltpu.sync_copy(rowbuf, acc_sh.at[dst_v.at[j]], add=True)
        pltpu.sync_copy(exbuf, den_sh.at[dst_v.at[j]], add=True)
        return 0

    lax.fori_loop(0, nblk, _blk, 0)
    plsc.subcore_barrier()

    for t in range(rpt // B):
        pltpu.sync_copy(acc_sh.at[pl.ds(base + t * B, B)],
                        acc_hbm.at[cid, pl.ds(base + t * B, B)])
    pltpu.sync_copy(den_sh.at[pl.ds(base, rpt)],
                    den_hbm.at[cid, pl.ds(base, rpt)])


def _sc_edge_pass(h, sn, edges_r, d):
    nblk = edges_r.shape[2]
    rpt = NP // NS
    mesh = plsc.VectorSubcoreMesh(
        core_axis_name="c", subcore_axis_name="s", num_cores=NC,
        num_subcores=NS)
    fn = pl.kernel(
        functools.partial(_sc_edge_body, nblk, d, rpt),
        out_type=[
            jax.ShapeDtypeStruct((NC, NP, d), jnp.float32),
            jax.ShapeDtypeStruct((NC, NP), jnp.float32),
        ],
        mesh=mesh,
        compiler_params=pltpu.CompilerParams(
            needs_layout_passes=False, use_tc_tiling_on_sc=False),
        scratch_types=[
            pltpu.VMEM((nblk, B), jnp.int32),
            pltpu.VMEM((nblk, B), jnp.int32),
            pltpu.VMEM((B, d), jnp.float32),
            pltpu.VMEM((B,), jnp.float32),
            pltpu.VMEM((B,), jnp.float32),
            pltpu.VMEM((B,), jnp.float32),
            pltpu.VMEM((B,), jnp.float32),
            pltpu.VMEM((rpt,), jnp.float32),
            pltpu.VMEM_SHARED((h.shape[0],), jnp.float32),
            pltpu.VMEM_SHARED((h.shape[0],), jnp.float32),
            pltpu.VMEM_SHARED((h.shape[0],), jnp.float32),
            pltpu.VMEM_SHARED((NP, d), jnp.float32),
            pltpu.VMEM_SHARED((NP,), jnp.float32),
            pltpu.SemaphoreType.DMA,
        ],
    )
    return fn(h, sn, edges_r)


def kernel(x, edge_index, W1, a1_src, a1_dst, b1, W2, a2_src, a2_dst, b2,
           W3, a3_src, a3_dst, b3):
    n = x.shape[0]
    e = edge_index.shape[1]
    ept = e // NW
    nblk = ept // B
    edges_r = edge_index.reshape(2, NW, nblk, B)

    h1, sn1 = _tc_pre(x, W1, a1_src, a1_dst)
    acc1, den1 = _sc_edge_pass(h1, sn1.T, edges_r, W1.shape[1])
    h2, sn2 = _tc_mid(acc1[:, :n], den1[:, :n, None], h1, sn1, b1,
                      W2, a2_src, a2_dst)
    acc2, den2 = _sc_edge_pass(h2, sn2.T, edges_r, W2.shape[1])
    h3, sn3 = _tc_mid(acc2[:, :n], den2[:, :n, None], h2, sn2, b2,
                      W3, a3_src, a3_dst)
    acc3, den3 = _sc_edge_pass(h3, sn3.T, edges_r, W3.shape[1])
    return _tc_final(acc3[:, :n], den3[:, :n, None], h3, sn3, b3)


# speedup vs baseline: 51.0329x; 1.3024x over previous
import functools

import jax
import jax.numpy as jnp
from jax import lax
from jax.experimental import pallas as pl
from jax.experimental.pallas import tpu as pltpu
from jax.experimental.pallas import tpu_sc as plsc

NC = 2
NS = 16
NW = NC * NS
B = 80
NP = 10240

_LEAKY = 0.2


def _leaky(v):
    return jnp.maximum(v, v * _LEAKY)


def _tc_pre_body(x_ref, w_ref, asrc_ref, adst_ref, h_ref, sn_ref):
    h = jnp.dot(x_ref[...], w_ref[...], preferred_element_type=jnp.float32)
    h_ref[...] = h
    a_s = jnp.dot(h, asrc_ref[...], preferred_element_type=jnp.float32)
    a_d = jnp.dot(h, adst_ref[...], preferred_element_type=jnp.float32)
    amax = jnp.max(a_s)
    c = _leaky(amax + a_d)
    sn_ref[...] = jnp.concatenate(
        [a_s, a_d, c, jnp.zeros_like(a_s)], axis=1)


def _tc_pre(x, w, a_src, a_dst):
    n = x.shape[0]
    dn = w.shape[1]
    return pl.pallas_call(
        _tc_pre_body,
        out_shape=[
            jax.ShapeDtypeStruct((n, dn), jnp.float32),
            jax.ShapeDtypeStruct((n, 4), jnp.float32),
        ],
    )(x, w, a_src.reshape(dn, 1), a_dst.reshape(dn, 1))


def _tc_norm(acc_ref, den_ref, h_ref, sn_ref, b_ref):
    a_s = sn_ref[:, 0:1]
    a_d = sn_ref[:, 1:2]
    c = sn_ref[:, 2:3]
    exs = jnp.exp(_leaky(a_s + a_d) - c)
    num = acc_ref[0] + acc_ref[1] + exs * h_ref[...]
    dent = den_ref[0] + den_ref[1] + exs + 1e-16
    return num / dent + b_ref[...]


def _tc_mid_body(acc_ref, den_ref, h_ref, sn_ref, b_ref, w_ref, asrc_ref,
                 adst_ref, h2_ref, sn2_ref):
    o = _tc_norm(acc_ref, den_ref, h_ref, sn_ref, b_ref)
    g = o * jax.nn.sigmoid(o)
    h2 = jnp.dot(g, w_ref[...], preferred_element_type=jnp.float32)
    h2_ref[...] = h2
    a_s = jnp.dot(h2, asrc_ref[...], preferred_element_type=jnp.float32)
    a_d = jnp.dot(h2, adst_ref[...], preferred_element_type=jnp.float32)
    amax = jnp.max(a_s)
    c = _leaky(amax + a_d)
    sn2_ref[...] = jnp.concatenate(
        [a_s, a_d, c, jnp.zeros_like(a_s)], axis=1)


def _tc_mid(acc, den, h, sn, b, w, a_src, a_dst):
    n, d = h.shape
    dn = w.shape[1]
    return pl.pallas_call(
        _tc_mid_body,
        out_shape=[
            jax.ShapeDtypeStruct((n, dn), jnp.float32),
            jax.ShapeDtypeStruct((n, 4), jnp.float32),
        ],
    )(acc, den, h, sn, b.reshape(1, d), w,
      a_src.reshape(dn, 1), a_dst.reshape(dn, 1))


def _tc_final_body(acc_ref, den_ref, h_ref, sn_ref, b_ref, out_ref):
    o = _tc_norm(acc_ref, den_ref, h_ref, sn_ref, b_ref)
    mx = jnp.max(o, axis=1, keepdims=True)
    lse = jnp.log(jnp.sum(jnp.exp(o - mx), axis=1, keepdims=True)) + mx
    out_ref[...] = o - lse


def _tc_final(acc, den, h, sn, b):
    n, d = h.shape
    return pl.pallas_call(
        _tc_final_body,
        out_shape=jax.ShapeDtypeStruct((n, d), jnp.float32),
    )(acc, den, h, sn, b.reshape(1, d))


def _sc_edge_body(nblk, d, rpt, h_hbm, sn_hbm, edges_hbm, acc_hbm, den_hbm,
                  src_v, dst_v, rowbuf, rowbuf1, asb, adb, cb, exbuf, zden_v,
                  as_sh, ad_sh, c_sh, acc_sh, den_sh, sem, sem1):
    cid = lax.axis_index("c")
    sid = lax.axis_index("s")
    wid = cid * NS + sid
    base = sid * rpt

    @pl.when(sid == 0)
    def _stage_tables():
        pltpu.sync_copy(sn_hbm.at[0], as_sh)
        pltpu.sync_copy(sn_hbm.at[1], ad_sh)
        pltpu.sync_copy(sn_hbm.at[2], c_sh)

    pltpu.sync_copy(edges_hbm.at[0, wid], src_v)
    pltpu.sync_copy(edges_hbm.at[1, wid], dst_v)

    zero16 = jnp.zeros((16,), jnp.float32)

    def _zrow(r, _):
        for q in range(d // 16):
            rowbuf[r, pl.ds(q * 16, 16)] = zero16
        return 0

    lax.fori_loop(0, B, _zrow, 0)

    def _zden(r, _):
        zden_v[pl.ds(r * 16, 16)] = zero16
        return 0

    lax.fori_loop(0, rpt // 16, _zden, 0)

    for t in range(rpt // B):
        pltpu.sync_copy(rowbuf, acc_sh.at[pl.ds(base + t * B, B)])
    pltpu.sync_copy(zden_v, den_sh.at[pl.ds(base, rpt)])
    plsc.subcore_barrier()

    def _gather(j, buf, s):
        pltpu.async_copy(h_hbm.at[src_v.at[j]], buf, s)

    def _gwait(buf, s):
        pltpu.make_async_copy(h_hbm.at[pl.ds(0, B)], buf, s).wait()

    def _process(j, buf):
        pltpu.sync_copy(as_sh.at[src_v.at[j]], asb)
        pltpu.sync_copy(ad_sh.at[dst_v.at[j]], adb)
        pltpu.sync_copy(c_sh.at[dst_v.at[j]], cb)

        for k in range(B // 16):
            as16 = asb[pl.ds(k * 16, 16)]
            ad16 = adb[pl.ds(k * 16, 16)]
            c16 = cb[pl.ds(k * 16, 16)]
            ex16 = jnp.exp(_leaky(as16 + ad16) - c16)
            exbuf[pl.ds(k * 16, 16)] = ex16

        def _scale(g, _):
            ex16 = exbuf[pl.ds(g * 16, 16)]
            for r in range(16):
                s = ex16[r]
                row = g * 16 + r
                for q in range(d // 16):
                    buf[row, pl.ds(q * 16, 16)] = (
                        buf[row, pl.ds(q * 16, 16)] * s)
            return 0

        lax.fori_loop(0, B // 16, _scale, 0)

        pltpu.sync_copy(buf, acc_sh.at[dst_v.at[j]], add=True)
        pltpu.sync_copy(exbuf, den_sh.at[dst_v.at[j]], add=True)

    _gather(0, rowbuf, sem)
    _gather(1, rowbuf1, sem1)

    def _pair(t, _):
        a = t * 2
        _gwait(rowbuf, sem)
        _process(a, rowbuf)

        @pl.when(a + 2 < nblk)
        def _():
            _gather(a + 2, rowbuf, sem)

        b = a + 1
        _gwait(rowbuf1, sem1)
        _process(b, rowbuf1)

        @pl.when(b + 2 < nblk)
        def _():
            _gather(b + 2, rowbuf1, sem1)

        return 0

    lax.fori_loop(0, nblk // 2, _pair, 0)
    if nblk % 2:
        _gwait(rowbuf, sem)
        _process(nblk - 1, rowbuf)
    plsc.subcore_barrier()

    for t in range(rpt // B):
        pltpu.sync_copy(acc_sh.at[pl.ds(base + t * B, B)],
                        acc_hbm.at[cid, pl.ds(base + t * B, B)])
    pltpu.sync_copy(den_sh.at[pl.ds(base, rpt)],
                    den_hbm.at[cid, pl.ds(base, rpt)])


def _sc_edge_pass(h, sn, edges_r, d):
    nblk = edges_r.shape[2]
    rpt = NP // NS
    mesh = plsc.VectorSubcoreMesh(
        core_axis_name="c", subcore_axis_name="s", num_cores=NC,
        num_subcores=NS)
    fn = pl.kernel(
        functools.partial(_sc_edge_body, nblk, d, rpt),
        out_type=[
            jax.ShapeDtypeStruct((NC, NP, d), jnp.float32),
            jax.ShapeDtypeStruct((NC, NP), jnp.float32),
        ],
        mesh=mesh,
        compiler_params=pltpu.CompilerParams(
            needs_layout_passes=False, use_tc_tiling_on_sc=False),
        scratch_types=[
            pltpu.VMEM((nblk, B), jnp.int32),
            pltpu.VMEM((nblk, B), jnp.int32),
            pltpu.VMEM((B, d), jnp.float32),
            pltpu.VMEM((B, d), jnp.float32),
            pltpu.VMEM((B,), jnp.float32),
            pltpu.VMEM((B,), jnp.float32),
            pltpu.VMEM((B,), jnp.float32),
            pltpu.VMEM((B,), jnp.float32),
            pltpu.VMEM((rpt,), jnp.float32),
            pltpu.VMEM_SHARED((h.shape[0],), jnp.float32),
            pltpu.VMEM_SHARED((h.shape[0],), jnp.float32),
            pltpu.VMEM_SHARED((h.shape[0],), jnp.float32),
            pltpu.VMEM_SHARED((NP, d), jnp.float32),
            pltpu.VMEM_SHARED((NP,), jnp.float32),
            pltpu.SemaphoreType.DMA,
            pltpu.SemaphoreType.DMA,
        ],
    )
    return fn(h, sn, edges_r)


def kernel(x, edge_index, W1, a1_src, a1_dst, b1, W2, a2_src, a2_dst, b2,
           W3, a3_src, a3_dst, b3):
    n = x.shape[0]
    e = edge_index.shape[1]
    ept = e // NW
    nblk = ept // B
    edges_r = edge_index.reshape(2, NW, nblk, B)

    h1, sn1 = _tc_pre(x, W1, a1_src, a1_dst)
    acc1, den1 = _sc_edge_pass(h1, sn1.T, edges_r, W1.shape[1])
    h2, sn2 = _tc_mid(acc1[:, :n], den1[:, :n, None], h1, sn1, b1,
                      W2, a2_src, a2_dst)
    acc2, den2 = _sc_edge_pass(h2, sn2.T, edges_r, W2.shape[1])
    h3, sn3 = _tc_mid(acc2[:, :n], den2[:, :n, None], h2, sn2, b2,
                      W3, a3_src, a3_dst)
    acc3, den3 = _sc_edge_pass(h3, sn3.T, edges_r, W3.shape[1])
    return _tc_final(acc3[:, :n], den3[:, :n, None], h3, sn3, b3)


# speedup vs baseline: 62.0685x; 1.2162x over previous
import functools

import jax
import jax.numpy as jnp
from jax import lax
from jax.experimental import pallas as pl
from jax.experimental.pallas import tpu as pltpu
from jax.experimental.pallas import tpu_sc as plsc

NC = 2
NS = 16
NW = NC * NS
B = 80
W = 25
NP = 10240

_LEAKY = 0.2


def _leaky(v):
    return jnp.maximum(v, v * _LEAKY)


def _tc_pre_body(x_ref, w_ref, asrc_ref, adst_ref, h_ref, sn_ref):
    h = jnp.dot(x_ref[...], w_ref[...], preferred_element_type=jnp.float32)
    h_ref[...] = h
    a_s = jnp.dot(h, asrc_ref[...], preferred_element_type=jnp.float32)
    a_d = jnp.dot(h, adst_ref[...], preferred_element_type=jnp.float32)
    amax = jnp.max(a_s)
    sn_ref[...] = jnp.concatenate(
        [a_s, a_d, jnp.full_like(a_s, amax), jnp.zeros_like(a_s)], axis=1)


def _tc_pre(x, w, a_src, a_dst):
    n = x.shape[0]
    dn = w.shape[1]
    return pl.pallas_call(
        _tc_pre_body,
        out_shape=[
            jax.ShapeDtypeStruct((n, dn), jnp.float32),
            jax.ShapeDtypeStruct((n, 4), jnp.float32),
        ],
    )(x, w, a_src.reshape(dn, 1), a_dst.reshape(dn, 1))


def _tc_norm(acc_ref, den_ref, h_ref, sn_ref, b_ref):
    a_s = sn_ref[:, 0:1]
    a_d = sn_ref[:, 1:2]
    c = _leaky(sn_ref[:, 2:3] + a_d)
    exs = jnp.exp(_leaky(a_s + a_d) - c)
    num = acc_ref[0] + acc_ref[1] + exs * h_ref[...]
    dent = den_ref[0] + den_ref[1] + exs + 1e-16
    return num / dent + b_ref[...]


def _tc_mid_body(acc_ref, den_ref, h_ref, sn_ref, b_ref, w_ref, asrc_ref,
                 adst_ref, h2_ref, sn2_ref):
    o = _tc_norm(acc_ref, den_ref, h_ref, sn_ref, b_ref)
    g = o * jax.nn.sigmoid(o)
    h2 = jnp.dot(g, w_ref[...], preferred_element_type=jnp.float32)
    h2_ref[...] = h2
    a_s = jnp.dot(h2, asrc_ref[...], preferred_element_type=jnp.float32)
    a_d = jnp.dot(h2, adst_ref[...], preferred_element_type=jnp.float32)
    amax = jnp.max(a_s)
    sn2_ref[...] = jnp.concatenate(
        [a_s, a_d, jnp.full_like(a_s, amax), jnp.zeros_like(a_s)], axis=1)


def _tc_mid(acc, den, h, sn, b, w, a_src, a_dst):
    n, d = h.shape
    dn = w.shape[1]
    return pl.pallas_call(
        _tc_mid_body,
        out_shape=[
            jax.ShapeDtypeStruct((n, dn), jnp.float32),
            jax.ShapeDtypeStruct((n, 4), jnp.float32),
        ],
    )(acc, den, h, sn, b.reshape(1, d), w,
      a_src.reshape(dn, 1), a_dst.reshape(dn, 1))


def _tc_final_body(acc_ref, den_ref, h_ref, sn_ref, b_ref, out_ref):
    o = _tc_norm(acc_ref, den_ref, h_ref, sn_ref, b_ref)
    mx = jnp.max(o, axis=1, keepdims=True)
    lse = jnp.log(jnp.sum(jnp.exp(o - mx), axis=1, keepdims=True)) + mx
    out_ref[...] = o - lse


def _tc_final(acc, den, h, sn, b):
    n, d = h.shape
    return pl.pallas_call(
        _tc_final_body,
        out_shape=jax.ShapeDtypeStruct((n, d), jnp.float32),
    )(acc, den, h, sn, b.reshape(1, d))


def _sc_edge_body(nblk, d, rpt, h_hbm, sn_hbm, edges_hbm, acc_hbm, den_hbm,
                  swin, dwin, rowbufs, exbufs, asb, adb, avec, zden_v,
                  as_sh, ad_sh, acc_sh, den_sh, gsems, ssems, wsem):
    cid = lax.axis_index("c")
    sid = lax.axis_index("s")
    wid = cid * NS + sid
    base = sid * rpt

    @pl.when(sid == 0)
    def _stage_tables():
        pltpu.sync_copy(sn_hbm.at[0], as_sh)
        pltpu.sync_copy(sn_hbm.at[1], ad_sh)

    pltpu.sync_copy(sn_hbm.at[2, pl.ds(0, 16)], avec)
    pltpu.sync_copy(edges_hbm.at[0, wid, pl.ds(0, W)], swin.at[0])
    pltpu.sync_copy(edges_hbm.at[1, wid, pl.ds(0, W)], dwin.at[0])

    zero16 = jnp.zeros((16,), jnp.float32)
    rb0 = rowbufs[0]

    def _zrow(r, _):
        for q in range(d // 16):
            rb0[r, pl.ds(q * 16, 16)] = zero16
        return 0

    lax.fori_loop(0, B, _zrow, 0)

    def _zden(r, _):
        zden_v[pl.ds(r * 16, 16)] = zero16
        return 0

    lax.fori_loop(0, rpt // 16, _zden, 0)

    for t in range(rpt // B):
        pltpu.sync_copy(rb0, acc_sh.at[pl.ds(base + t * B, B)])
    pltpu.sync_copy(zden_v, den_sh.at[pl.ds(base, rpt)])
    plsc.subcore_barrier()

    def _widx(j):
        return (j // W) % 2, j % W

    def _gather(j, p):
        pw, bo = _widx(j)
        pltpu.async_copy(h_hbm.at[swin.at[pw, bo]], rowbufs[p], gsems[p])

    def _gwait(p):
        pltpu.make_async_copy(
            h_hbm.at[pl.ds(0, B)], rowbufs[p], gsems[p]).wait()

    def _swait(p):
        pltpu.make_async_copy(
            rowbufs[p], acc_sh.at[pl.ds(0, B)], ssems[p]).wait()
        pltpu.make_async_copy(
            exbufs[p], den_sh.at[pl.ds(0, B)], ssems[p]).wait()

    def _compute(j, p):
        buf = rowbufs[p]
        exbuf = exbufs[p]
        amax = avec[pl.ds(0, 16)]

        pw, bo = _widx(j)
        pltpu.sync_copy(as_sh.at[swin.at[pw, bo]], asb)
        pltpu.sync_copy(ad_sh.at[dwin.at[pw, bo]], adb)

        for k in range(B // 16):
            as16 = asb[pl.ds(k * 16, 16)]
            ad16 = adb[pl.ds(k * 16, 16)]
            c16 = _leaky(amax + ad16)
            ex16 = jnp.exp(_leaky(as16 + ad16) - c16)
            exbuf[pl.ds(k * 16, 16)] = ex16

        def _scale(g, _):
            ex16 = exbuf[pl.ds(g * 16, 16)]
            for r in range(16):
                s = ex16[r]
                row = g * 16 + r
                for q in range(d // 16):
                    buf[row, pl.ds(q * 16, 16)] = (
                        buf[row, pl.ds(q * 16, 16)] * s)
            return 0

        lax.fori_loop(0, B // 16, _scale, 0)

    def _scatter(j, p):
        pw, bo = _widx(j)
        pltpu.async_copy(rowbufs[p], acc_sh.at[dwin.at[pw, bo]], ssems[p],
                         add=True)
        pltpu.async_copy(exbufs[p], den_sh.at[dwin.at[pw, bo]], ssems[p],
                         add=True)

    _gather(0, 0)
    _gather(1, 1)

    def _step(j, p):
        _gwait(p)
        _compute(j, p)
        _scatter(j, p)
        pn = (p + 2) % 3

        @pl.when(j >= 1)
        def _():
            _swait(pn)

        @pl.when(jnp.logical_and(j % W == W - 5, j + 5 < nblk))
        def _():
            wn = (j // W + 1) % (nblk // W)
            pltpu.async_copy(edges_hbm.at[0, wid, pl.ds(wn * W, W)],
                             swin.at[wn % 2], wsem)
            pltpu.async_copy(edges_hbm.at[1, wid, pl.ds(wn * W, W)],
                             dwin.at[wn % 2], wsem)

        @pl.when(jnp.logical_and(j % W == W - 2, j + 2 < nblk))
        def _():
            pltpu.make_async_copy(edges_hbm.at[0, 0, pl.ds(0, W)],
                                  swin.at[0], wsem).wait()
            pltpu.make_async_copy(edges_hbm.at[1, 0, pl.ds(0, W)],
                                  dwin.at[0], wsem).wait()

        @pl.when(j + 2 < nblk)
        def _():
            _gather(j + 2, pn)

    def _triple(t, _):
        j = t * 3
        _step(j, 0)
        _step(j + 1, 1)
        _step(j + 2, 2)
        return 0

    lax.fori_loop(0, nblk // 3, _triple, 0)
    for u in range(nblk - (nblk // 3) * 3):
        _step((nblk // 3) * 3 + u, u % 3)
    _swait((nblk - 1) % 3)
    plsc.subcore_barrier()

    for t in range(rpt // B):
        pltpu.sync_copy(acc_sh.at[pl.ds(base + t * B, B)],
                        acc_hbm.at[cid, pl.ds(base + t * B, B)])
    pltpu.sync_copy(den_sh.at[pl.ds(base, rpt)],
                    den_hbm.at[cid, pl.ds(base, rpt)])


def _sc_edge_pass(h, sn, edges_r, d):
    nblk = edges_r.shape[2]
    rpt = NP // NS
    mesh = plsc.VectorSubcoreMesh(
        core_axis_name="c", subcore_axis_name="s", num_cores=NC,
        num_subcores=NS)
    fn = pl.kernel(
        functools.partial(_sc_edge_body, nblk, d, rpt),
        out_type=[
            jax.ShapeDtypeStruct((NC, NP, d), jnp.float32),
            jax.ShapeDtypeStruct((NC, NP), jnp.float32),
        ],
        mesh=mesh,
        compiler_params=pltpu.CompilerParams(
            needs_layout_passes=False, use_tc_tiling_on_sc=False),
        scratch_types=[
            pltpu.VMEM((2, W, B), jnp.int32),
            pltpu.VMEM((2, W, B), jnp.int32),
            [pltpu.VMEM((B, d), jnp.float32)] * 3,
            [pltpu.VMEM((B,), jnp.float32)] * 3,
            pltpu.VMEM((B,), jnp.float32),
            pltpu.VMEM((B,), jnp.float32),
            pltpu.VMEM((16,), jnp.float32),
            pltpu.VMEM((rpt,), jnp.float32),
            pltpu.VMEM_SHARED((h.shape[0],), jnp.float32),
            pltpu.VMEM_SHARED((h.shape[0],), jnp.float32),
            pltpu.VMEM_SHARED((NP, d), jnp.float32),
            pltpu.VMEM_SHARED((NP,), jnp.float32),
            [pltpu.SemaphoreType.DMA] * 3,
            [pltpu.SemaphoreType.DMA] * 3,
            pltpu.SemaphoreType.DMA,
        ],
    )
    return fn(h, sn, edges_r)


def kernel(x, edge_index, W1, a1_src, a1_dst, b1, W2, a2_src, a2_dst, b2,
           W3, a3_src, a3_dst, b3):
    n = x.shape[0]
    e = edge_index.shape[1]
    ept = e // NW
    nblk = ept // B
    edges_r = edge_index.reshape(2, NW, nblk, B)

    h1, sn1 = _tc_pre(x, W1, a1_src, a1_dst)
    acc1, den1 = _sc_edge_pass(h1, sn1.T, edges_r, W1.shape[1])
    h2, sn2 = _tc_mid(acc1[:, :n], den1[:, :n, None], h1, sn1, b1,
                      W2, a2_src, a2_dst)
    acc2, den2 = _sc_edge_pass(h2, sn2.T, edges_r, W2.shape[1])
    h3, sn3 = _tc_mid(acc2[:, :n], den2[:, :n, None], h2, sn2, b2,
                      W3, a3_src, a3_dst)
    acc3, den3 = _sc_edge_pass(h3, sn3.T, edges_r, W3.shape[1])
    return _tc_final(acc3[:, :n], den3[:, :n, None], h3, sn3, b3)


# speedup vs baseline: 67.8554x; 1.0932x over previous
import functools

import jax
import jax.numpy as jnp
from jax import lax
from jax.experimental import pallas as pl
from jax.experimental.pallas import tpu as pltpu
from jax.experimental.pallas import tpu_sc as plsc

NC = 2
NS = 16
NW = NC * NS
B = 80
W = 25
NP = 10240

_LEAKY = 0.2


def _leaky(v):
    return jnp.maximum(v, v * _LEAKY)


def _tc_pre_body(x_ref, w_ref, asrc_ref, adst_ref, h_ref, sn_ref):
    h = jnp.dot(x_ref[...], w_ref[...], preferred_element_type=jnp.float32)
    h_ref[...] = h
    a_s = jnp.dot(h, asrc_ref[...], preferred_element_type=jnp.float32)
    a_d = jnp.dot(h, adst_ref[...], preferred_element_type=jnp.float32)
    amax = jnp.max(a_s)
    sn_ref[...] = jnp.concatenate(
        [a_s, a_d, jnp.full_like(a_s, amax), jnp.zeros_like(a_s)], axis=1)


def _tc_pre(x, w, a_src, a_dst):
    n = x.shape[0]
    dn = w.shape[1]
    return pl.pallas_call(
        _tc_pre_body,
        out_shape=[
            jax.ShapeDtypeStruct((n, dn), jnp.float32),
            jax.ShapeDtypeStruct((n, 4), jnp.float32),
        ],
    )(x, w, a_src.reshape(dn, 1), a_dst.reshape(dn, 1))


def _tc_norm(acc_ref, den_ref, h_ref, sn_ref, b_ref):
    a_s = sn_ref[:, 0:1]
    a_d = sn_ref[:, 1:2]
    c = _leaky(sn_ref[:, 2:3] + a_d)
    exs = jnp.exp(_leaky(a_s + a_d) - c)
    num = acc_ref[0] + acc_ref[1] + exs * h_ref[...]
    dent = den_ref[0] + den_ref[1] + exs + 1e-16
    return num / dent + b_ref[...]


def _tc_mid_body(acc_ref, den_ref, h_ref, sn_ref, b_ref, w_ref, asrc_ref,
                 adst_ref, h2_ref, sn2_ref):
    o = _tc_norm(acc_ref, den_ref, h_ref, sn_ref, b_ref)
    g = o * jax.nn.sigmoid(o)
    h2 = jnp.dot(g, w_ref[...], preferred_element_type=jnp.float32)
    h2_ref[...] = h2
    a_s = jnp.dot(h2, asrc_ref[...], preferred_element_type=jnp.float32)
    a_d = jnp.dot(h2, adst_ref[...], preferred_element_type=jnp.float32)
    amax = jnp.max(a_s)
    sn2_ref[...] = jnp.concatenate(
        [a_s, a_d, jnp.full_like(a_s, amax), jnp.zeros_like(a_s)], axis=1)


def _tc_mid(acc, den, h, sn, b, w, a_src, a_dst):
    n, d = h.shape
    dn = w.shape[1]
    return pl.pallas_call(
        _tc_mid_body,
        out_shape=[
            jax.ShapeDtypeStruct((n, dn), jnp.float32),
            jax.ShapeDtypeStruct((n, 4), jnp.float32),
        ],
    )(acc, den, h, sn, b.reshape(1, d), w,
      a_src.reshape(dn, 1), a_dst.reshape(dn, 1))


def _tc_final_body(acc_ref, den_ref, h_ref, sn_ref, b_ref, out_ref):
    o = _tc_norm(acc_ref, den_ref, h_ref, sn_ref, b_ref)
    mx = jnp.max(o, axis=1, keepdims=True)
    lse = jnp.log(jnp.sum(jnp.exp(o - mx), axis=1, keepdims=True)) + mx
    out_ref[...] = o - lse


def _tc_final(acc, den, h, sn, b):
    n, d = h.shape
    return pl.pallas_call(
        _tc_final_body,
        out_shape=jax.ShapeDtypeStruct((n, d), jnp.float32),
    )(acc, den, h, sn, b.reshape(1, d))


def _sc_edge_body(nblk, d, rpt, h_hbm, sn_hbm, edges_hbm, acc_hbm, den_hbm,
                  swin, dwin, rowbufs, exbufs, asbs, adbs, avec,
                  zden_v, as_sh, ad_sh, acc_sh, den_sh, gsems, ssems, wsem,
                  asems):
    cid = lax.axis_index("c")
    sid = lax.axis_index("s")
    wid = cid * NS + sid
    base = sid * rpt

    @pl.when(sid == 0)
    def _stage_tables():
        pltpu.sync_copy(sn_hbm.at[0], as_sh)
        pltpu.sync_copy(sn_hbm.at[1], ad_sh)

    pltpu.sync_copy(sn_hbm.at[2, pl.ds(0, 16)], avec)
    pltpu.sync_copy(edges_hbm.at[0, wid, pl.ds(0, W)], swin.at[0])
    pltpu.sync_copy(edges_hbm.at[1, wid, pl.ds(0, W)], dwin.at[0])

    zero16 = jnp.zeros((16,), jnp.float32)
    rb0 = rowbufs[0]

    def _zrow(r, _):
        for q in range(d // 16):
            rb0[r, pl.ds(q * 16, 16)] = zero16
        return 0

    lax.fori_loop(0, B, _zrow, 0)

    def _zden(r, _):
        zden_v[pl.ds(r * 16, 16)] = zero16
        return 0

    lax.fori_loop(0, rpt // 16, _zden, 0)

    for t in range(rpt // B):
        pltpu.sync_copy(rb0, acc_sh.at[pl.ds(base + t * B, B)])
    pltpu.sync_copy(zden_v, den_sh.at[pl.ds(base, rpt)])
    plsc.subcore_barrier()

    def _widx(j):
        return (j // W) % 2, j % W

    def _gather(j, p):
        pw, bo = _widx(j)
        pltpu.async_copy(h_hbm.at[swin.at[pw, bo]], rowbufs[p], gsems[p])

    def _gwait(p):
        pltpu.make_async_copy(
            h_hbm.at[pl.ds(0, B)], rowbufs[p], gsems[p]).wait()

    def _swait(p):
        pltpu.make_async_copy(
            rowbufs[p], acc_sh.at[pl.ds(0, B)], ssems[p]).wait()
        pltpu.make_async_copy(
            exbufs[p], den_sh.at[pl.ds(0, B)], ssems[p]).wait()

    def _scal_issue(j, q):
        pw, bo = _widx(j)
        pltpu.async_copy(as_sh.at[swin.at[pw, bo]], asbs[q], asems[q])
        pltpu.async_copy(ad_sh.at[dwin.at[pw, bo]], adbs[q], asems[q])

    def _scal_wait(q):
        pltpu.make_async_copy(as_sh.at[pl.ds(0, B)], asbs[q], asems[q]).wait()
        pltpu.make_async_copy(ad_sh.at[pl.ds(0, B)], adbs[q], asems[q]).wait()

    def _compute(j, p, q):
        buf = rowbufs[p]
        exbuf = exbufs[p]
        asb = asbs[q]
        adb = adbs[q]
        amax = avec[pl.ds(0, 16)]

        for k in range(B // 16):
            as16 = asb[pl.ds(k * 16, 16)]
            ad16 = adb[pl.ds(k * 16, 16)]
            c16 = _leaky(amax + ad16)
            ex16 = jnp.exp(_leaky(as16 + ad16) - c16)
            exbuf[pl.ds(k * 16, 16)] = ex16

        def _scale(g, _):
            ex16 = exbuf[pl.ds(g * 16, 16)]
            for r in range(16):
                s = ex16[r]
                row = g * 16 + r
                for q in range(d // 16):
                    buf[row, pl.ds(q * 16, 16)] = (
                        buf[row, pl.ds(q * 16, 16)] * s)
            return 0

        lax.fori_loop(0, B // 16, _scale, 0)

    def _scatter(j, p):
        pw, bo = _widx(j)
        pltpu.async_copy(rowbufs[p], acc_sh.at[dwin.at[pw, bo]], ssems[p],
                         add=True)
        pltpu.async_copy(exbufs[p], den_sh.at[dwin.at[pw, bo]], ssems[p],
                         add=True)

    _gather(0, 0)
    _gather(1, 1)
    _scal_issue(0, 0)

    def _step(j, p):
        _gwait(p)
        _scal_wait(p)

        @pl.when(j + 1 < nblk)
        def _():
            _scal_issue(j + 1, (p + 1) % 3)

        _compute(j, p, p)
        _scatter(j, p)
        pn = (p + 2) % 3

        @pl.when(j >= 1)
        def _():
            _swait(pn)

        @pl.when(jnp.logical_and(j % W == W - 5, j + 5 < nblk))
        def _():
            wn = (j // W + 1) % (nblk // W)
            pltpu.async_copy(edges_hbm.at[0, wid, pl.ds(wn * W, W)],
                             swin.at[wn % 2], wsem)
            pltpu.async_copy(edges_hbm.at[1, wid, pl.ds(wn * W, W)],
                             dwin.at[wn % 2], wsem)

        @pl.when(jnp.logical_and(j % W == W - 2, j + 2 < nblk))
        def _():
            pltpu.make_async_copy(edges_hbm.at[0, 0, pl.ds(0, W)],
                                  swin.at[0], wsem).wait()
            pltpu.make_async_copy(edges_hbm.at[1, 0, pl.ds(0, W)],
                                  dwin.at[0], wsem).wait()

        @pl.when(j + 2 < nblk)
        def _():
            _gather(j + 2, pn)

    def _triple(t, _):
        j = t * 3
        _step(j, 0)
        _step(j + 1, 1)
        _step(j + 2, 2)
        return 0

    lax.fori_loop(0, nblk // 3, _triple, 0)
    for u in range(nblk - (nblk // 3) * 3):
        _step((nblk // 3) * 3 + u, u % 3)
    _swait((nblk - 1) % 3)
    plsc.subcore_barrier()

    for t in range(rpt // B):
        pltpu.sync_copy(acc_sh.at[pl.ds(base + t * B, B)],
                        acc_hbm.at[cid, pl.ds(base + t * B, B)])
    pltpu.sync_copy(den_sh.at[pl.ds(base, rpt)],
                    den_hbm.at[cid, pl.ds(base, rpt)])


def _sc_edge_pass(h, sn, edges_r, d):
    nblk = edges_r.shape[2]
    rpt = NP // NS
    mesh = plsc.VectorSubcoreMesh(
        core_axis_name="c", subcore_axis_name="s", num_cores=NC,
        num_subcores=NS)
    fn = pl.kernel(
        functools.partial(_sc_edge_body, nblk, d, rpt),
        out_type=[
            jax.ShapeDtypeStruct((NC, NP, d), jnp.float32),
            jax.ShapeDtypeStruct((NC, NP), jnp.float32),
        ],
        mesh=mesh,
        compiler_params=pltpu.CompilerParams(
            needs_layout_passes=False, use_tc_tiling_on_sc=False),
        scratch_types=[
            pltpu.VMEM((2, W, B), jnp.int32),
            pltpu.VMEM((2, W, B), jnp.int32),
            [pltpu.VMEM((B, d), jnp.float32)] * 3,
            [pltpu.VMEM((B,), jnp.float32)] * 3,
            [pltpu.VMEM((B,), jnp.float32)] * 3,
            [pltpu.VMEM((B,), jnp.float32)] * 3,
            pltpu.VMEM((16,), jnp.float32),
            pltpu.VMEM((rpt,), jnp.float32),
            pltpu.VMEM_SHARED((h.shape[0],), jnp.float32),
            pltpu.VMEM_SHARED((h.shape[0],), jnp.float32),
            pltpu.VMEM_SHARED((NP, d), jnp.float32),
            pltpu.VMEM_SHARED((NP,), jnp.float32),
            [pltpu.SemaphoreType.DMA] * 3,
            [pltpu.SemaphoreType.DMA] * 3,
            pltpu.SemaphoreType.DMA,
            [pltpu.SemaphoreType.DMA] * 3,
        ],
    )
    return fn(h, sn, edges_r)


def kernel(x, edge_index, W1, a1_src, a1_dst, b1, W2, a2_src, a2_dst, b2,
           W3, a3_src, a3_dst, b3):
    n = x.shape[0]
    e = edge_index.shape[1]
    ept = e // NW
    nblk = ept // B
    edges_r = edge_index.reshape(2, NW, nblk, B)

    xp = jnp.concatenate(
        [x, jnp.zeros((NP - n, x.shape[1]), x.dtype)], axis=0)

    h1, sn1 = _tc_pre(xp, W1, a1_src, a1_dst)
    acc1, den1 = _sc_edge_pass(h1, sn1.T, edges_r, W1.shape[1])
    h2, sn2 = _tc_mid(acc1, den1[:, :, None], h1, sn1, b1,
                      W2, a2_src, a2_dst)
    acc2, den2 = _sc_edge_pass(h2, sn2.T, edges_r, W2.shape[1])
    h3, sn3 = _tc_mid(acc2, den2[:, :, None], h2, sn2, b2,
                      W3, a3_src, a3_dst)
    acc3, den3 = _sc_edge_pass(h3, sn3.T, edges_r, W3.shape[1])
    return _tc_final(acc3, den3[:, :, None], h3, sn3, b3)[:n]


# speedup vs baseline: 67.9824x; 1.0019x over previous
import functools

import jax
import jax.numpy as jnp
from jax import lax
from jax.experimental import pallas as pl
from jax.experimental.pallas import tpu as pltpu
from jax.experimental.pallas import tpu_sc as plsc

NC = 2
NS = 16
NW = NC * NS
B = 80
W = 25
NP = 10240

_LEAKY = 0.2


def _leaky(v):
    return jnp.maximum(v, v * _LEAKY)


def _tc_pre_body(x_ref, w_ref, asrc_ref, adst_ref, h_ref, sn_ref):
    h = jnp.dot(x_ref[...], w_ref[...], preferred_element_type=jnp.float32)
    h_ref[...] = h
    a_s = jnp.dot(h, asrc_ref[...], preferred_element_type=jnp.float32)
    a_d = jnp.dot(h, adst_ref[...], preferred_element_type=jnp.float32)
    amax = jnp.max(a_s)
    sn_ref[...] = jnp.concatenate(
        [a_s, a_d, jnp.full_like(a_s, amax), jnp.zeros_like(a_s)], axis=1)


def _tc_pre(x, w, a_src, a_dst):
    n = x.shape[0]
    dn = w.shape[1]
    return pl.pallas_call(
        _tc_pre_body,
        compiler_params=pltpu.CompilerParams(skip_device_barrier=True),
        out_shape=[
            jax.ShapeDtypeStruct((n, dn), jnp.float32),
            jax.ShapeDtypeStruct((n, 4), jnp.float32),
        ],
    )(x, w, a_src.reshape(dn, 1), a_dst.reshape(dn, 1))


def _tc_norm(acc_ref, den_ref, h_ref, sn_ref, b_ref):
    a_s = sn_ref[:, 0:1]
    a_d = sn_ref[:, 1:2]
    c = _leaky(sn_ref[:, 2:3] + a_d)
    exs = jnp.exp(_leaky(a_s + a_d) - c)
    num = acc_ref[0] + acc_ref[1] + exs * h_ref[...]
    dent = den_ref[0] + den_ref[1] + exs + 1e-16
    return num / dent + b_ref[...]


def _tc_mid_body(acc_ref, den_ref, h_ref, sn_ref, b_ref, w_ref, asrc_ref,
                 adst_ref, h2_ref, sn2_ref):
    o = _tc_norm(acc_ref, den_ref, h_ref, sn_ref, b_ref)
    g = o * jax.nn.sigmoid(o)
    h2 = jnp.dot(g, w_ref[...], preferred_element_type=jnp.float32)
    h2_ref[...] = h2
    a_s = jnp.dot(h2, asrc_ref[...], preferred_element_type=jnp.float32)
    a_d = jnp.dot(h2, adst_ref[...], preferred_element_type=jnp.float32)
    amax = jnp.max(a_s)
    sn2_ref[...] = jnp.concatenate(
        [a_s, a_d, jnp.full_like(a_s, amax), jnp.zeros_like(a_s)], axis=1)


def _tc_mid(acc, den, h, sn, b, w, a_src, a_dst):
    n, d = h.shape
    dn = w.shape[1]
    return pl.pallas_call(
        _tc_mid_body,
        compiler_params=pltpu.CompilerParams(skip_device_barrier=True),
        out_shape=[
            jax.ShapeDtypeStruct((n, dn), jnp.float32),
            jax.ShapeDtypeStruct((n, 4), jnp.float32),
        ],
    )(acc, den, h, sn, b.reshape(1, d), w,
      a_src.reshape(dn, 1), a_dst.reshape(dn, 1))


def _tc_final_body(acc_ref, den_ref, h_ref, sn_ref, b_ref, out_ref):
    o = _tc_norm(acc_ref, den_ref, h_ref, sn_ref, b_ref)
    mx = jnp.max(o, axis=1, keepdims=True)
    lse = jnp.log(jnp.sum(jnp.exp(o - mx), axis=1, keepdims=True)) + mx
    out_ref[...] = o - lse


def _tc_final(acc, den, h, sn, b):
    n, d = h.shape
    return pl.pallas_call(
        _tc_final_body,
        compiler_params=pltpu.CompilerParams(skip_device_barrier=True),
        out_shape=jax.ShapeDtypeStruct((n, d), jnp.float32),
    )(acc, den, h, sn, b.reshape(1, d))


def _sc_edge_body(nblk, d, rpt, h_hbm, sn_hbm, edges_hbm, acc_hbm, den_hbm,
                  swin, dwin, rowbufs, exbufs, asbs, adbs, avec,
                  zden_v, as_sh, ad_sh, acc_sh, den_sh, gsems, ssems, wsem,
                  asems):
    cid = lax.axis_index("c")
    sid = lax.axis_index("s")
    wid = cid * NS + sid
    base = sid * rpt

    @pl.when(sid == 0)
    def _stage_tables():
        pltpu.sync_copy(sn_hbm.at[0], as_sh)
        pltpu.sync_copy(sn_hbm.at[1], ad_sh)

    pltpu.sync_copy(sn_hbm.at[2, pl.ds(0, 16)], avec)
    pltpu.sync_copy(edges_hbm.at[0, wid, pl.ds(0, W)], swin.at[0])
    pltpu.sync_copy(edges_hbm.at[1, wid, pl.ds(0, W)], dwin.at[0])

    zero16 = jnp.zeros((16,), jnp.float32)
    rb0 = rowbufs[0]

    def _zrow(r, _):
        for q in range(d // 16):
            rb0[r, pl.ds(q * 16, 16)] = zero16
        return 0

    lax.fori_loop(0, B, _zrow, 0)

    def _zden(r, _):
        zden_v[pl.ds(r * 16, 16)] = zero16
        return 0

    lax.fori_loop(0, rpt // 16, _zden, 0)

    for t in range(rpt // B):
        pltpu.sync_copy(rb0, acc_sh.at[pl.ds(base + t * B, B)])
    pltpu.sync_copy(zden_v, den_sh.at[pl.ds(base, rpt)])
    plsc.subcore_barrier()

    def _widx(j):
        return (j // W) % 2, j % W

    def _gather(j, p):
        pw, bo = _widx(j)
        pltpu.async_copy(h_hbm.at[swin.at[pw, bo]], rowbufs[p], gsems[p])

    def _gwait(p):
        pltpu.make_async_copy(
            h_hbm.at[pl.ds(0, B)], rowbufs[p], gsems[p]).wait()

    def _swait(p):
        pltpu.make_async_copy(
            rowbufs[p], acc_sh.at[pl.ds(0, B)], ssems[p]).wait()
        pltpu.make_async_copy(
            exbufs[p], den_sh.at[pl.ds(0, B)], ssems[p]).wait()

    def _scal_issue(j, q):
        pw, bo = _widx(j)
        pltpu.async_copy(as_sh.at[swin.at[pw, bo]], asbs[q], asems[q])
        pltpu.async_copy(ad_sh.at[dwin.at[pw, bo]], adbs[q], asems[q])

    def _scal_wait(q):
        pltpu.make_async_copy(as_sh.at[pl.ds(0, B)], asbs[q], asems[q]).wait()
        pltpu.make_async_copy(ad_sh.at[pl.ds(0, B)], adbs[q], asems[q]).wait()

    def _compute(j, p, q):
        buf = rowbufs[p]
        exbuf = exbufs[p]
        asb = asbs[q]
        adb = adbs[q]
        amax = avec[pl.ds(0, 16)]

        for k in range(B // 16):
            as16 = asb[pl.ds(k * 16, 16)]
            ad16 = adb[pl.ds(k * 16, 16)]
            c16 = _leaky(amax + ad16)
            ex16 = jnp.exp(_leaky(as16 + ad16) - c16)
            exbuf[pl.ds(k * 16, 16)] = ex16

        def _scale(g, _):
            ex16 = exbuf[pl.ds(g * 16, 16)]
            for r in range(16):
                s = ex16[r]
                row = g * 16 + r
                for q in range(d // 16):
                    buf[row, pl.ds(q * 16, 16)] = (
                        buf[row, pl.ds(q * 16, 16)] * s)
            return 0

        lax.fori_loop(0, B // 16, _scale, 0)

    def _scatter(j, p):
        pw, bo = _widx(j)
        pltpu.async_copy(rowbufs[p], acc_sh.at[dwin.at[pw, bo]], ssems[p],
                         add=True)
        pltpu.async_copy(exbufs[p], den_sh.at[dwin.at[pw, bo]], ssems[p],
                         add=True)

    _gather(0, 0)
    _gather(1, 1)
    _scal_issue(0, 0)

    def _step(j, p):
        _gwait(p)
        _scal_wait(p)

        @pl.when(j + 1 < nblk)
        def _():
            _scal_issue(j + 1, (p + 1) % 3)

        _compute(j, p, p)
        _scatter(j, p)
        pn = (p + 2) % 3

        @pl.when(j >= 1)
        def _():
            _swait(pn)

        @pl.when(jnp.logical_and(j % W == W - 5, j + 5 < nblk))
        def _():
            wn = (j // W + 1) % (nblk // W)
            pltpu.async_copy(edges_hbm.at[0, wid, pl.ds(wn * W, W)],
                             swin.at[wn % 2], wsem)
            pltpu.async_copy(edges_hbm.at[1, wid, pl.ds(wn * W, W)],
                             dwin.at[wn % 2], wsem)

        @pl.when(jnp.logical_and(j % W == W - 2, j + 2 < nblk))
        def _():
            pltpu.make_async_copy(edges_hbm.at[0, 0, pl.ds(0, W)],
                                  swin.at[0], wsem).wait()
            pltpu.make_async_copy(edges_hbm.at[1, 0, pl.ds(0, W)],
                                  dwin.at[0], wsem).wait()

        @pl.when(j + 2 < nblk)
        def _():
            _gather(j + 2, pn)

    def _triple(t, _):
        j = t * 3
        _step(j, 0)
        _step(j + 1, 1)
        _step(j + 2, 2)
        return 0

    lax.fori_loop(0, nblk // 3, _triple, 0)
    for u in range(nblk - (nblk // 3) * 3):
        _step((nblk // 3) * 3 + u, u % 3)
    _swait((nblk - 1) % 3)
    plsc.subcore_barrier()

    for t in range(rpt // B):
        pltpu.sync_copy(acc_sh.at[pl.ds(base + t * B, B)],
                        acc_hbm.at[cid, pl.ds(base + t * B, B)])
    pltpu.sync_copy(den_sh.at[pl.ds(base, rpt)],
                    den_hbm.at[cid, pl.ds(base, rpt)])


def _sc_edge_pass(h, sn, edges_r, d):
    nblk = edges_r.shape[2]
    rpt = NP // NS
    mesh = plsc.VectorSubcoreMesh(
        core_axis_name="c", subcore_axis_name="s", num_cores=NC,
        num_subcores=NS)
    fn = pl.kernel(
        functools.partial(_sc_edge_body, nblk, d, rpt),
        out_type=[
            jax.ShapeDtypeStruct((NC, NP, d), jnp.float32),
            jax.ShapeDtypeStruct((NC, NP), jnp.float32),
        ],
        mesh=mesh,
        compiler_params=pltpu.CompilerParams(
            needs_layout_passes=False, use_tc_tiling_on_sc=False,
            skip_device_barrier=True),
        scratch_types=[
            pltpu.VMEM((2, W, B), jnp.int32),
            pltpu.VMEM((2, W, B), jnp.int32),
            [pltpu.VMEM((B, d), jnp.float32)] * 3,
            [pltpu.VMEM((B,), jnp.float32)] * 3,
            [pltpu.VMEM((B,), jnp.float32)] * 3,
            [pltpu.VMEM((B,), jnp.float32)] * 3,
            pltpu.VMEM((16,), jnp.float32),
            pltpu.VMEM((rpt,), jnp.float32),
            pltpu.VMEM_SHARED((h.shape[0],), jnp.float32),
            pltpu.VMEM_SHARED((h.shape[0],), jnp.float32),
            pltpu.VMEM_SHARED((NP, d), jnp.float32),
            pltpu.VMEM_SHARED((NP,), jnp.float32),
            [pltpu.SemaphoreType.DMA] * 3,
            [pltpu.SemaphoreType.DMA] * 3,
            pltpu.SemaphoreType.DMA,
            [pltpu.SemaphoreType.DMA] * 3,
        ],
    )
    return fn(h, sn, edges_r)


def kernel(x, edge_index, W1, a1_src, a1_dst, b1, W2, a2_src, a2_dst, b2,
           W3, a3_src, a3_dst, b3):
    n = x.shape[0]
    e = edge_index.shape[1]
    ept = e // NW
    nblk = ept // B
    edges_r = edge_index.reshape(2, NW, nblk, B)

    xp = jnp.concatenate(
        [x, jnp.zeros((NP - n, x.shape[1]), x.dtype)], axis=0)

    h1, sn1 = _tc_pre(xp, W1, a1_src, a1_dst)
    acc1, den1 = _sc_edge_pass(h1, sn1.T, edges_r, W1.shape[1])
    h2, sn2 = _tc_mid(acc1, den1[:, :, None], h1, sn1, b1,
                      W2, a2_src, a2_dst)
    acc2, den2 = _sc_edge_pass(h2, sn2.T, edges_r, W2.shape[1])
    h3, sn3 = _tc_mid(acc2, den2[:, :, None], h2, sn2, b2,
                      W3, a3_src, a3_dst)
    acc3, den3 = _sc_edge_pass(h3, sn3.T, edges_r, W3.shape[1])
    return _tc_final(acc3, den3[:, :, None], h3, sn3, b3)[:n]
